# baseline (speedup 1.0000x reference)
import functools

import jax
import jax.numpy as jnp
from jax import lax
from jax.experimental import pallas as pl
from jax.experimental.pallas import tpu as pltpu
from jax.experimental.pallas import tpu_sc as plsc

B = 100000
D = 200
V = 120
H = 128
NC = 2
NS = 16
NW = NC * NS
C = 128
NFULL = B // C
TAIL = B - NFULL * C
TAIL_WORKER = NFULL % NW
NKMAX = -(-NFULL // NW)
NB = 2

_mesh = plsc.VectorSubcoreMesh(core_axis_name="c", subcore_axis_name="s")


@functools.partial(
    pl.kernel,
    mesh=_mesh,
    compiler_params=pltpu.CompilerParams(needs_layout_passes=False),
    out_type=jax.ShapeDtypeStruct((B, D), jnp.float32),
    scratch_types=[
        pltpu.VMEM((NKMAX, C), jnp.int32),
        pltpu.VMEM((V, H), jnp.float32),
        pltpu.VMEM((V, H), jnp.float32),
        pltpu.VMEM((NB, C, H), jnp.float32),
        pltpu.VMEM((NB, C, H), jnp.float32),
        pltpu.SemaphoreType.DMA,
        pltpu.SemaphoreType.DMA,
    ],
)
def _gather_kernel(idx_hbm, ta_hbm, tb_hbm, out_hbm, idx_v, ta_v, tb_v,
                   ra_v, rb_v, sem_i, sem_s):
    wid = lax.axis_index("s") * NC + lax.axis_index("c")
    nk = (NFULL - wid + NW - 1) // NW
    lanes = jnp.arange(16, dtype=jnp.int32)
    h_dyn = wid * 0 + H

    pltpu.sync_copy(ta_hbm, ta_v)
    pltpu.sync_copy(tb_hbm, tb_v)

    def chunk_base(k):
        return (wid + k * NW) * C

    def fire_idx(k, carry):
        pltpu.async_copy(idx_hbm.at[pl.ds(chunk_base(k), C)], idx_v.at[k], sem_i)
        return carry
    lax.fori_loop(0, nk, fire_idx, 0)

    @pl.when(wid == TAIL_WORKER)
    def _():
        pltpu.sync_copy(idx_hbm.at[pl.ds(NFULL * C, TAIL)],
                        idx_v.at[NKMAX - 1, pl.ds(0, TAIL)])

    def expand(k, buf, ngroups):
        rvecs = [idx_v[k, pl.ds(g * 16, 16)] for g in range(ngroups)]

        def col_body(c, carry):
            colv = lax.bitwise_and(c + lanes, H - 1)
            for g in range(len(rvecs)):
                rowv = lanes + (g * 16)
                a = plsc.load_gather(ta_v, [carry[g], colv])
                plsc.store_scatter(ra_v.at[buf], [rowv, colv], a)
                b = plsc.load_gather(tb_v, [carry[g], colv])
                plsc.store_scatter(rb_v.at[buf], [rowv, colv], b)
            return carry

        lax.fori_loop(0, H, col_body, tuple(rvecs))

    def store_issue(k, buf):
        pltpu.async_copy(ra_v.at[buf],
                         out_hbm.at[pl.ds(chunk_base(k), C), pl.ds(0, H)],
                         sem_s)
        pltpu.async_copy(rb_v.at[buf],
                         out_hbm.at[pl.ds(chunk_base(k), C), pl.ds(h_dyn, H)],
                         sem_s)

    def store_wait():
        pltpu.make_async_copy(ra_v.at[0],
                              out_hbm.at[pl.ds(0, C), pl.ds(0, H)],
                              sem_s).wait()

    def body(k, carry):
        buf = lax.rem(k, NB)

        @pl.when(k >= NB)
        def _():
            store_wait()
            store_wait()

        pltpu.make_async_copy(idx_hbm.at[pl.ds(0, C)], idx_v.at[0],
                              sem_i).wait()
        expand(k, buf, C // 16)
        store_issue(k, buf)
        return carry

    lax.fori_loop(0, nk, body, 0)

    def drain(j, carry):
        store_wait()
        store_wait()
        return carry
    lax.fori_loop(0, lax.min(nk, NB), drain, 0)

    @pl.when(wid == TAIL_WORKER)
    def _():
        expand(NKMAX - 1, 0, TAIL // 16)
        pltpu.sync_copy(ra_v.at[0, pl.ds(0, TAIL)],
                        out_hbm.at[pl.ds(NFULL * C, TAIL), pl.ds(0, H)])
        pltpu.sync_copy(rb_v.at[0, pl.ds(0, TAIL)],
                        out_hbm.at[pl.ds(NFULL * C, TAIL), pl.ds(h_dyn, H)])


def kernel(x, atom_fea):
    ta = atom_fea[:, :H]
    tb = jnp.pad(atom_fea[:, H:], ((0, 0), (0, 2 * H - D)))
    return _gather_kernel(x.astype(jnp.int32), ta, tb)


# speedup vs baseline: 1.9693x; 1.9693x over previous
import functools

import jax
import jax.numpy as jnp
from jax import lax
from jax.experimental import pallas as pl
from jax.experimental.pallas import tpu as pltpu
from jax.experimental.pallas import tpu_sc as plsc

B = 100000
D = 200
V = 120
H = 128
NC = 2
NS = 16
NW = NC * NS
C = 128
NFULL = B // C
TAIL = B - NFULL * C
TAIL_WORKER = NFULL % NW
NKMAX = -(-NFULL // NW)
NB = 2

_mesh = plsc.VectorSubcoreMesh(core_axis_name="c", subcore_axis_name="s")


@functools.partial(
    pl.kernel,
    mesh=_mesh,
    compiler_params=pltpu.CompilerParams(needs_layout_passes=False),
    out_type=jax.ShapeDtypeStruct((B, D), jnp.float32),
    scratch_types=[
        pltpu.VMEM((NKMAX, C), jnp.int32),
        pltpu.VMEM((V, H), jnp.float32),
        pltpu.VMEM((V, H), jnp.float32),
        pltpu.VMEM((NB, C, H), jnp.float32),
        pltpu.VMEM((NB, C, H), jnp.float32),
        pltpu.SemaphoreType.DMA,
        pltpu.SemaphoreType.DMA,
    ],
)
def _gather_kernel(idx_hbm, ta_hbm, tb_hbm, out_hbm, idx_v, ta_v, tb_v,
                   ra_v, rb_v, sem_i, sem_s):
    wid = lax.axis_index("s") * NC + lax.axis_index("c")
    nk = (NFULL - wid + NW - 1) // NW
    lanes = jnp.arange(16, dtype=jnp.int32)
    h_dyn = wid * 0 + H

    pltpu.sync_copy(ta_hbm, ta_v)
    pltpu.sync_copy(tb_hbm, tb_v)

    def chunk_base(k):
        return (wid + k * NW) * C

    def fire_idx(k, carry):
        pltpu.async_copy(idx_hbm.at[pl.ds(chunk_base(k), C)], idx_v.at[k], sem_i)
        return carry
    lax.fori_loop(0, nk, fire_idx, 0)

    @pl.when(wid == TAIL_WORKER)
    def _():
        pltpu.sync_copy(idx_hbm.at[pl.ds(NFULL * C, TAIL)],
                        idx_v.at[NKMAX - 1, pl.ds(0, TAIL)])

    def expand(k, buf, ngroups):
        rvecs = [idx_v[k, pl.ds(g * 16, 16)] for g in range(ngroups)]

        @plsc.parallel_loop(0, H, unroll=4, carry=tuple(rvecs))
        def col_body(c, carry):
            colv = lax.bitwise_and(c + lanes, H - 1)
            for g in range(len(rvecs)):
                rowv = lanes + (g * 16)
                a = plsc.load_gather(ta_v, [carry[g], colv])
                plsc.store_scatter(ra_v.at[buf], [rowv, colv], a)
                b = plsc.load_gather(tb_v, [carry[g], colv])
                plsc.store_scatter(rb_v.at[buf], [rowv, colv], b)
            return carry

    def store_issue(k, buf):
        pltpu.async_copy(ra_v.at[buf],
                         out_hbm.at[pl.ds(chunk_base(k), C), pl.ds(0, H)],
                         sem_s)
        pltpu.async_copy(rb_v.at[buf],
                         out_hbm.at[pl.ds(chunk_base(k), C), pl.ds(h_dyn, H)],
                         sem_s)

    def store_wait():
        pltpu.make_async_copy(ra_v.at[0],
                              out_hbm.at[pl.ds(0, C), pl.ds(0, H)],
                              sem_s).wait()

    def body(k, carry):
        buf = lax.rem(k, NB)

        @pl.when(k >= NB)
        def _():
            store_wait()
            store_wait()

        pltpu.make_async_copy(idx_hbm.at[pl.ds(0, C)], idx_v.at[0],
                              sem_i).wait()
        expand(k, buf, C // 16)
        store_issue(k, buf)
        return carry

    lax.fori_loop(0, nk, body, 0)

    def drain(j, carry):
        store_wait()
        store_wait()
        return carry
    lax.fori_loop(0, lax.min(nk, NB), drain, 0)

    @pl.when(wid == TAIL_WORKER)
    def _():
        expand(NKMAX - 1, 0, TAIL // 16)
        pltpu.sync_copy(ra_v.at[0, pl.ds(0, TAIL)],
                        out_hbm.at[pl.ds(NFULL * C, TAIL), pl.ds(0, H)])
        pltpu.sync_copy(rb_v.at[0, pl.ds(0, TAIL)],
                        out_hbm.at[pl.ds(NFULL * C, TAIL), pl.ds(h_dyn, H)])


def kernel(x, atom_fea):
    ta = atom_fea[:, :H]
    tb = jnp.pad(atom_fea[:, H:], ((0, 0), (0, 2 * H - D)))
    return _gather_kernel(x.astype(jnp.int32), ta, tb)


# speedup vs baseline: 1.9756x; 1.0032x over previous
import functools

import jax
import jax.numpy as jnp
from jax import lax
from jax.experimental import pallas as pl
from jax.experimental.pallas import tpu as pltpu
from jax.experimental.pallas import tpu_sc as plsc

B = 100000
D = 200
V = 120
H = 128
NC = 2
NS = 16
NW = NC * NS
C = 128
NFULL = B // C
TAIL = B - NFULL * C
TAIL_WORKER = NW - 1
NKMAX = -(-NFULL // NW)
REM = NFULL - (NKMAX - 1) * NW
NB = 2

_mesh = plsc.VectorSubcoreMesh(core_axis_name="c", subcore_axis_name="s")


@functools.partial(
    pl.kernel,
    mesh=_mesh,
    compiler_params=pltpu.CompilerParams(needs_layout_passes=False),
    out_type=jax.ShapeDtypeStruct((B, D), jnp.float32),
    scratch_types=[
        pltpu.VMEM((NKMAX * C,), jnp.int32),
        pltpu.VMEM((V, H), jnp.float32),
        pltpu.VMEM((V, H), jnp.float32),
        pltpu.VMEM((NB, C, H), jnp.float32),
        pltpu.VMEM((NB, C, H), jnp.float32),
        pltpu.SemaphoreType.DMA,
        pltpu.SemaphoreType.DMA,
    ],
)
def _gather_kernel(idx_hbm, ta_hbm, tb_hbm, out_hbm, idx_v, ta_v, tb_v,
                   ra_v, rb_v, sem_i, sem_s):
    wid = lax.axis_index("s") * NC + lax.axis_index("c")
    nk = jnp.where(wid < REM, NKMAX, NKMAX - 1)
    start = jnp.where(wid < REM, wid * NKMAX,
                      REM * NKMAX + (wid - REM) * (NKMAX - 1))
    lanes = jnp.arange(16, dtype=jnp.int32)
    h_dyn = wid * 0 + H

    pltpu.sync_copy(ta_hbm, ta_v)
    pltpu.sync_copy(tb_hbm, tb_v)

    def chunk_base(k):
        return (start + k) * C

    @pl.when(wid < REM)
    def _():
        pltpu.async_copy(idx_hbm.at[pl.ds(start * C, NKMAX * C)],
                         idx_v.at[pl.ds(0, NKMAX * C)], sem_i)

    @pl.when(wid >= REM)
    def _():
        pltpu.async_copy(idx_hbm.at[pl.ds(start * C, (NKMAX - 1) * C)],
                         idx_v.at[pl.ds(0, (NKMAX - 1) * C)], sem_i)

    @pl.when(wid == TAIL_WORKER)
    def _():
        pltpu.sync_copy(idx_hbm.at[pl.ds(NFULL * C, TAIL)],
                        idx_v.at[pl.ds((NKMAX - 1) * C, TAIL)])

    def expand(k, buf, ngroups):
        rvecs = [idx_v[pl.ds(k * C + g * 16, 16)] for g in range(ngroups)]

        @plsc.parallel_loop(0, H, unroll=8, carry=tuple(rvecs))
        def col_body(c, carry):
            colv = lax.bitwise_and(c + lanes, H - 1)
            for g in range(len(rvecs)):
                rowv = lanes + (g * 16)
                a = plsc.load_gather(ta_v, [carry[g], colv])
                plsc.store_scatter(ra_v.at[buf], [rowv, colv], a)
                b = plsc.load_gather(tb_v, [carry[g], colv])
                plsc.store_scatter(rb_v.at[buf], [rowv, colv], b)
            return carry

    def store_issue(k, buf):
        pltpu.async_copy(ra_v.at[buf],
                         out_hbm.at[pl.ds(chunk_base(k), C), pl.ds(0, H)],
                         sem_s)
        pltpu.async_copy(rb_v.at[buf],
                         out_hbm.at[pl.ds(chunk_base(k), C), pl.ds(h_dyn, H)],
                         sem_s)

    def store_wait():
        pltpu.make_async_copy(ra_v.at[0],
                              out_hbm.at[pl.ds(0, C), pl.ds(0, H)],
                              sem_s).wait()

    @pl.when(wid < REM)
    def _():
        pltpu.make_async_copy(idx_hbm.at[pl.ds(0, NKMAX * C)],
                              idx_v.at[pl.ds(0, NKMAX * C)], sem_i).wait()

    @pl.when(wid >= REM)
    def _():
        pltpu.make_async_copy(idx_hbm.at[pl.ds(0, (NKMAX - 1) * C)],
                              idx_v.at[pl.ds(0, (NKMAX - 1) * C)], sem_i).wait()

    def body(k, carry):
        buf = lax.rem(k, NB)

        @pl.when(k >= NB)
        def _():
            store_wait()
            store_wait()

        expand(k, buf, C // 16)
        store_issue(k, buf)
        return carry

    lax.fori_loop(0, nk, body, 0)

    def drain(j, carry):
        store_wait()
        store_wait()
        return carry
    lax.fori_loop(0, lax.min(nk, NB), drain, 0)

    @pl.when(wid == TAIL_WORKER)
    def _():
        expand(NKMAX - 1, 0, TAIL // 16)
        pltpu.sync_copy(ra_v.at[0, pl.ds(0, TAIL)],
                        out_hbm.at[pl.ds(NFULL * C, TAIL), pl.ds(0, H)])
        pltpu.sync_copy(rb_v.at[0, pl.ds(0, TAIL)],
                        out_hbm.at[pl.ds(NFULL * C, TAIL), pl.ds(h_dyn, H)])


def kernel(x, atom_fea):
    ta = atom_fea[:, :H]
    tb = jnp.pad(atom_fea[:, H:], ((0, 0), (0, 2 * H - D)))
    return _gather_kernel(x.astype(jnp.int32), ta, tb)
